# Initial kernel scaffold; baseline (speedup 1.0000x reference)
import jax
import jax.numpy as jnp
from jax.experimental import pallas as pl


def kernel(x, edge_idx, W1, b1, W2, b2, W3, b3, W4, b4):
    raise NotImplementedError("write your pallas kernel here")


# speedup vs baseline: 9.2093x; 9.2093x over previous
import functools

import jax
import jax.numpy as jnp
from jax import lax
from jax.experimental import pallas as pl
from jax.experimental.pallas import tpu as pltpu
from jax.experimental.pallas import tpu_sc as plsc

N_NODES = 10000
N_EDGES = 160000
D = 256
QW = D // 4
NC, NS = 2, 16
EPT = N_EDGES // NS
CHUNK = 80
NCHUNK = EPT // CHUNK
STRIPE = 624
ZROWS = 208
DEGW = 16


def _one_pass(x_q, e_unused, agg_sp, deg_sp, agg_out, deg_out, do_deg,
              src_v, dst_v, rows_v, ones_v, zrow_v, zdeg_v, sem, tid):
    for k in range(STRIPE // ZROWS):
        base = tid * STRIPE + k * ZROWS
        pltpu.sync_copy(zrow_v, agg_sp.at[pl.ds(base, ZROWS), :])

        @pl.when(do_deg)
        def _():
            pltpu.sync_copy(zdeg_v, deg_sp.at[pl.ds(base, ZROWS), :])

    @pl.when(tid == NS - 1)
    def _():
        pltpu.sync_copy(zrow_v.at[pl.ds(0, 16), :],
                        agg_sp.at[pl.ds(N_NODES - 16, 16), :])

        @pl.when(do_deg)
        def _():
            pltpu.sync_copy(zdeg_v.at[pl.ds(0, 16), :],
                            deg_sp.at[pl.ds(N_NODES - 16, 16), :])

    plsc.subcore_barrier()

    def step(j, _):
        src_j = src_v.at[j]
        dst_j = dst_v.at[j]
        pltpu.async_copy(x_q.at[src_j], rows_v, sem).wait()
        pltpu.sync_copy(rows_v, agg_sp.at[dst_j], add=True)

        @pl.when(do_deg)
        def _():
            pltpu.sync_copy(ones_v, deg_sp.at[dst_j], add=True)
        return 0

    lax.fori_loop(0, NCHUNK, step, 0)
    plsc.subcore_barrier()

    def writeout(sp, hbm):
        base = tid * STRIPE
        pltpu.sync_copy(sp.at[pl.ds(base, STRIPE), :],
                        hbm.at[pl.ds(base, STRIPE), :])

        @pl.when(tid == NS - 1)
        def _():
            pltpu.sync_copy(sp.at[pl.ds(N_NODES - 16, 16), :],
                            hbm.at[pl.ds(N_NODES - 16, 16), :])

    writeout(agg_sp, agg_out)

    @pl.when(do_deg)
    def _():
        writeout(deg_sp, deg_out)

    plsc.subcore_barrier()


def _sc_aggregate_body(xq0, xq1, xq2, xq3, e_ref,
                       agg0, agg1, agg2, agg3, deg_out,
                       src_v, dst_v, rows_v, ones_v, zrow_v, zdeg_v,
                       agg_sp, deg_sp, sem):
    cid = lax.axis_index("c")
    tid = lax.axis_index("s")

    pltpu.sync_copy(e_ref.at[0, tid], src_v)
    pltpu.sync_copy(e_ref.at[1, tid], dst_v)

    def zinit(i, _):
        for j in range(QW // 16):
            zrow_v[i, pl.ds(j * 16, 16)] = jnp.zeros((16,), jnp.float32)
        zdeg_v[i, :] = jnp.zeros((16,), jnp.float32)

        @pl.when(i < CHUNK)
        def _():
            ones_v[i, :] = jnp.ones((16,), jnp.float32)
        return 0

    lax.fori_loop(0, ZROWS, zinit, 0)

    quarters = ((xq0, agg0), (xq1, agg1), (xq2, agg2), (xq3, agg3))
    for p in range(2):
        for c in range(NC):
            x_q, agg_out = quarters[2 * p + c]
            do_deg = (p == 0 and c == 0)

            @pl.when(cid == c)
            def _(x_q=x_q, agg_out=agg_out, do_deg=do_deg):
                _one_pass(x_q, e_ref, agg_sp, deg_sp, agg_out, deg_out,
                          do_deg, src_v, dst_v, rows_v, ones_v, zrow_v,
                          zdeg_v, sem, tid)


@functools.cache
def _make_sc_aggregate():
    aggq = jax.ShapeDtypeStruct((N_NODES, QW), jnp.float32)
    return pl.kernel(
        _sc_aggregate_body,
        out_type=(aggq, aggq, aggq, aggq,
                  jax.ShapeDtypeStruct((N_NODES, DEGW), jnp.float32)),
        mesh=plsc.VectorSubcoreMesh(
            core_axis_name="c", subcore_axis_name="s", num_cores=NC,
            num_subcores=NS),
        scratch_types=(
            pltpu.VMEM((NCHUNK, CHUNK), jnp.int32),
            pltpu.VMEM((NCHUNK, CHUNK), jnp.int32),
            pltpu.VMEM((CHUNK, QW), jnp.float32),
            pltpu.VMEM((CHUNK, DEGW), jnp.float32),
            pltpu.VMEM((ZROWS, QW), jnp.float32),
            pltpu.VMEM((ZROWS, DEGW), jnp.float32),
            pltpu.VMEM_SHARED((N_NODES, QW), jnp.float32),
            pltpu.VMEM_SHARED((N_NODES, DEGW), jnp.float32),
            pltpu.SemaphoreType.DMA,
        ),
        compiler_params=pltpu.CompilerParams(use_tc_tiling_on_sc=False),
    )


BR = 1000


def _tc_kernel(x_ref, q0_ref, q1_ref, q2_ref, q3_ref, deg_ref,
               w1_ref, b1_ref, w2_ref, b2_ref, w3_ref, b3_ref,
               w4_ref, b4_ref, out_ref):
    dn = (((1,), (1,)), ((), ()))
    f32 = jnp.float32
    x1 = lax.dot_general(x_ref[...], w1_ref[...], dn,
                         preferred_element_type=f32) + b1_ref[...]
    agg = jnp.concatenate(
        [q0_ref[...], q1_ref[...], q2_ref[...], q3_ref[...]], axis=1)
    x2 = lax.dot_general(agg, w2_ref[...], dn, preferred_element_type=f32)
    s = lax.dot_general(agg, w3_ref[...] + w4_ref[...], dn,
                        preferred_element_type=f32)
    deg2 = 2.0 * deg_ref[:, 0:1]
    x2 = 2.0 * x2 + deg2 * b2_ref[...]
    s = 2.0 * s + deg2 * (b3_ref[...] + b4_ref[...])
    out_ref[...] = x1 + jax.nn.sigmoid(s) * x2


def _tc_combine(x, aggs, deg, W1, b1, W2, b2, W3, b3, W4, b4):
    grid = (N_NODES // BR,)
    row_spec = lambda w: pl.BlockSpec((BR, w), lambda i: (i, 0))
    full = lambda a, b: pl.BlockSpec((a, b), lambda i: (0, 0))
    return pl.pallas_call(
        _tc_kernel,
        grid=grid,
        in_specs=[
            row_spec(D), row_spec(QW), row_spec(QW), row_spec(QW),
            row_spec(QW), row_spec(DEGW),
            full(D, D), full(1, D), full(D, D), full(1, D),
            full(D, D), full(1, D), full(D, D), full(1, D),
        ],
        out_specs=row_spec(D),
        out_shape=jax.ShapeDtypeStruct((N_NODES, D), jnp.float32),
    )(x, *aggs, deg, W1, b1, W2, b2, W3, b3, W4, b4)


def kernel(x, edge_idx, W1, b1, W2, b2, W3, b3, W4, b4):
    xq = [x[:, i * QW:(i + 1) * QW] for i in range(4)]
    e_r = edge_idx.astype(jnp.int32).reshape(2, NS, NCHUNK, CHUNK)
    q0, q1, q2, q3, deg = _make_sc_aggregate()(*xq, e_r)
    return _tc_combine(x, (q0, q1, q2, q3), deg,
                       W1, b1.reshape(1, D), W2, b2.reshape(1, D),
                       W3, b3.reshape(1, D), W4, b4.reshape(1, D))
